# baseline (speedup 1.0000x reference)
import jax
import jax.numpy as jnp
from jax import lax
from jax.experimental import pallas as pl
from jax.experimental.pallas import tpu as pltpu
from jax.experimental.pallas import tpu_sc as plsc

N = 10000
E = 320000
NC, NS = 2, 16
NW = NC * NS
CH = 128
NCH = 79
PH0, PH1 = 40, 39
NP = 10112
RPT = NP // NS
BR = 632
mesh = plsc.VectorSubcoreMesh(core_axis_name="c", subcore_axis_name="s")


def _make_agg(width, with_counts):
  def fill(rows_v, val):
    def vstep(r, carry):
      for cc in range(8):
        rows_v[0, r, pl.ds(cc * 16, 16)] = jnp.full((16,), val, jnp.float32)
      return carry

    lax.fori_loop(0, CH, vstep, 0)

  def zero_acc(rows_v, acc_sh, s):
    def zstep(k, carry):
      pltpu.sync_copy(rows_v.at[0],
                      acc_sh.at[pl.ds(s * RPT + k * CH, CH)])
      return carry

    lax.fori_loop(0, RPT // CH, zstep, 0)
    rem = RPT % CH
    pltpu.sync_copy(rows_v.at[0, pl.ds(0, rem)],
                    acc_sh.at[pl.ds(s * RPT + (RPT // CH) * CH, rem)])

  def body(y_hbm, edge_hbm, out_hbm, *rest):
    if with_counts:
      cnt_hbm, idx_v, rows_v, acc_sh, sem = rest
    else:
      idx_v, rows_v, acc_sh, sem = rest
    c = lax.axis_index("c")
    s = lax.axis_index("s")
    wid = c * NS + s

    fill(rows_v, 0.0)
    zero_acc(rows_v, acc_sh, s)
    pltpu.sync_copy(edge_hbm.at[wid, :, pl.ds(0, PH0)], idx_v)
    src_v = idx_v.at[0]
    dst_v = idx_v.at[1]
    plsc.subcore_barrier()

    def run_phase(nch):
      pltpu.async_copy(y_hbm.at[src_v.at[0]], rows_v.at[0], sem)

      def step(j, carry):
        p = lax.rem(j, 2)
        q = 1 - p
        pltpu.make_async_copy(y_hbm.at[src_v.at[j]], rows_v.at[p],
                              sem).wait()

        @pl.when(j + 1 < nch)
        def _():
          pltpu.async_copy(y_hbm.at[src_v.at[j + 1]], rows_v.at[q], sem)

        pltpu.sync_copy(rows_v.at[p], acc_sh.at[dst_v.at[j]], add=True)
        return carry

      lax.fori_loop(0, nch, step, 0)

    run_phase(PH0)
    pltpu.sync_copy(edge_hbm.at[wid, :, pl.ds(PH0, PH1)],
                    idx_v.at[:, pl.ds(0, PH1)])
    run_phase(PH1)

    plsc.subcore_barrier()
    pltpu.sync_copy(acc_sh.at[pl.ds(s * RPT, RPT)],
                    out_hbm.at[pl.ds(c * NP + s * RPT, RPT)])

    if with_counts:
      plsc.subcore_barrier()
      fill(rows_v, 0.0)
      zero_acc(rows_v, acc_sh, s)
      fill(rows_v, 1.0)
      plsc.subcore_barrier()

      def cstep(nch):
        def cs(j, carry):
          pltpu.sync_copy(rows_v.at[0], acc_sh.at[idx_v.at[1].at[j]],
                          add=True)
          return carry

        lax.fori_loop(0, nch, cs, 0)

      pltpu.sync_copy(edge_hbm.at[wid, :, pl.ds(0, PH0)], idx_v)
      cstep(PH0)
      pltpu.sync_copy(edge_hbm.at[wid, :, pl.ds(PH0, PH1)],
                      idx_v.at[:, pl.ds(0, PH1)])
      cstep(PH1)
      plsc.subcore_barrier()
      pltpu.sync_copy(acc_sh.at[pl.ds(s * RPT, RPT)],
                      cnt_hbm.at[pl.ds(c * NP + s * RPT, RPT)])

  out_type = [jax.ShapeDtypeStruct((NC * NP, width), jnp.float32)]
  if with_counts:
    out_type.append(jax.ShapeDtypeStruct((NC * NP, width), jnp.float32))
  return pl.kernel(
      body,
      out_type=out_type,
      mesh=mesh,
      scratch_types=[
          pltpu.VMEM((2, PH0, CH), jnp.int32),
          pltpu.VMEM((2, CH, width), jnp.float32),
          pltpu.VMEM_SHARED((NP, width), jnp.float32),
          pltpu.SemaphoreType.DMA,
      ])


_agg128c = _make_agg(128, True)
_agg128 = _make_agg(128, False)


def _pre_body(x_ref, wl_ref, wr_ref, b_ref, y_ref, r_ref):
  xb = x_ref[...]
  y_ref[...] = jnp.dot(xb, wl_ref[...], preferred_element_type=jnp.float32)
  r_ref[...] = (jnp.dot(xb, wr_ref[...], preferred_element_type=jnp.float32)
                + b_ref[...])


def _mid_body(p_ref, c_ref, r1_ref, wr_ref, b_ref, h_ref, r_ref, cm_ref):
  cm = jnp.maximum(c_ref[0, :, 0:1] + c_ref[1, :, 0:1], 1.0)
  cm_ref[...] = cm
  mean = (p_ref[0] + p_ref[1]) / cm
  h = jnp.maximum(mean + r1_ref[...], 0.0)
  h_ref[...] = h
  r_ref[...] = (jnp.dot(h, wr_ref[...], preferred_element_type=jnp.float32)
                + b_ref[...])


def _post_body(p_ref, cm_ref, r2_ref, wl_ref, wt_ref, b_ref, o_ref):
  mean = (p_ref[0] + p_ref[1]) / cm_ref[...]
  agg = jnp.dot(mean, wl_ref[...], preferred_element_type=jnp.float32)
  h = jnp.maximum(agg + r2_ref[...], 0.0)
  l0 = jnp.sum(h * wt_ref[0:1, :], axis=1, keepdims=True) + b_ref[0, 0]
  l1 = jnp.sum(h * wt_ref[1:2, :], axis=1, keepdims=True) + b_ref[0, 1]
  m = jnp.maximum(l0, l1)
  lse = m + jnp.log(jnp.exp(l0 - m) + jnp.exp(l1 - m))
  o_ref[...] = jnp.concatenate([l0 - lse, l1 - lse], axis=1)


def _full(shape):
  return pl.BlockSpec(shape, lambda i: (0,) * len(shape))


def _rows(shape):
  return pl.BlockSpec(shape, lambda i: (i,) + (0,) * (len(shape) - 1))


def _pre(x, wl, wr, b):
  return pl.pallas_call(
      _pre_body,
      grid=(NP // BR,),
      in_specs=[_rows((BR, 128)), _full((128, 128)), _full((128, 128)),
                _full((1, 128))],
      out_specs=[_rows((BR, 128)), _rows((BR, 128))],
      out_shape=[jax.ShapeDtypeStruct((NP, 128), jnp.float32)] * 2,
  )(x, wl, wr, b)


def _mid(p, cnt, r1, wr, b):
  return pl.pallas_call(
      _mid_body,
      grid=(NP // BR,),
      in_specs=[pl.BlockSpec((2, BR, 128), lambda i: (0, i, 0)),
                pl.BlockSpec((2, BR, 128), lambda i: (0, i, 0)),
                _rows((BR, 128)), _full((128, 64)), _full((1, 64))],
      out_specs=[_rows((BR, 128)), _rows((BR, 64)), _rows((BR, 1))],
      out_shape=[jax.ShapeDtypeStruct((NP, 128), jnp.float32),
                 jax.ShapeDtypeStruct((NP, 64), jnp.float32),
                 jax.ShapeDtypeStruct((NP, 1), jnp.float32)],
  )(p, cnt, r1, wr, b)


def _post(p, cm, r2, wl, wt, b):
  return pl.pallas_call(
      _post_body,
      grid=(NP // BR,),
      in_specs=[pl.BlockSpec((2, BR, 128), lambda i: (0, i, 0)),
                _rows((BR, 1)), _rows((BR, 64)), _full((128, 64)),
                _full((2, 64)), _full((1, 2))],
      out_specs=_rows((BR, 2)),
      out_shape=jax.ShapeDtypeStruct((NP, 2), jnp.float32),
  )(p, cm, r2, wl, wt, b)


def kernel(x, edge_index, W1_l, b1, W1_r, W2_l, b2, W2_r, W_lin, b_lin):
  srcw = edge_index[0].astype(jnp.int32).reshape(NW, E // NW)
  dstw = edge_index[1].astype(jnp.int32).reshape(NW, E // NW)
  pad = NCH * CH - E // NW
  src3 = jnp.pad(srcw, ((0, 0), (0, pad))).reshape(NW, NCH, CH)
  dst3 = jnp.pad(dstw, ((0, 0), (0, pad)),
                 constant_values=NP - 1).reshape(NW, NCH, CH)
  edge3 = jnp.stack([src3, dst3], axis=1)
  x_p = jnp.pad(x, ((0, NP - N), (0, 0)))

  y1, r1 = _pre(x_p, W1_l, W1_r, b1.reshape(1, -1))
  p1, cnt = _agg128c(y1, edge3)
  p1 = p1.reshape(NC, NP, 128)
  cnt = cnt.reshape(NC, NP, 128)
  h1, r2, cm = _mid(p1, cnt, r1, W2_r, b2.reshape(1, -1))
  (p2,) = _agg128(h1, edge3)
  p2 = p2.reshape(NC, NP, 128)
  out = _post(p2, cm, r2, W2_l, W_lin.T, b_lin.reshape(1, -1))
  return out[:N]


# speedup vs baseline: 1.4640x; 1.4640x over previous
import jax
import jax.numpy as jnp
from jax import lax
from jax.experimental import pallas as pl
from jax.experimental.pallas import tpu as pltpu
from jax.experimental.pallas import tpu_sc as plsc

N = 10000
E = 320000
NC, NS = 2, 16
NW = NC * NS
CH = 80
NCH = E // NW // CH
PH0, PH1 = 64, 61
NP = 10240
RPT = NP // NS
BR = 1024
mesh = plsc.VectorSubcoreMesh(core_axis_name="c", subcore_axis_name="s")


def _make_agg(width, with_counts):
  def fill(rows_v, val):
    def vstep(r, carry):
      for cc in range(8):
        rows_v[0, r, pl.ds(cc * 16, 16)] = jnp.full((16,), val, jnp.float32)
      return carry

    lax.fori_loop(0, CH, vstep, 0)

  def zero_acc(rows_v, acc_sh, s):
    def zstep(k, carry):
      pltpu.sync_copy(rows_v.at[0],
                      acc_sh.at[pl.ds(s * RPT + k * CH, CH)])
      return carry

    lax.fori_loop(0, RPT // CH, zstep, 0)

  def body(y_hbm, edge_hbm, out_hbm, *rest):
    if with_counts:
      cnt_hbm, idx_v, rows_v, acc_sh, sem = rest
    else:
      idx_v, rows_v, acc_sh, sem = rest
    c = lax.axis_index("c")
    s = lax.axis_index("s")
    wid = c * NS + s

    fill(rows_v, 0.0)
    zero_acc(rows_v, acc_sh, s)
    pltpu.sync_copy(edge_hbm.at[wid, :, pl.ds(0, PH0)], idx_v)
    src_v = idx_v.at[0]
    dst_v = idx_v.at[1]
    plsc.subcore_barrier()

    def run_phase(nch):
      pltpu.async_copy(y_hbm.at[src_v.at[0]], rows_v.at[0], sem)

      def step(j, carry):
        p = lax.rem(j, 2)
        q = 1 - p
        pltpu.make_async_copy(y_hbm.at[src_v.at[j]], rows_v.at[p],
                              sem).wait()

        @pl.when(j + 1 < nch)
        def _():
          pltpu.async_copy(y_hbm.at[src_v.at[j + 1]], rows_v.at[q], sem)

        pltpu.sync_copy(rows_v.at[p], acc_sh.at[dst_v.at[j]], add=True)
        return carry

      lax.fori_loop(0, nch, step, 0)

    run_phase(PH0)
    pltpu.sync_copy(edge_hbm.at[wid, :, pl.ds(PH0, PH1)],
                    idx_v.at[:, pl.ds(0, PH1)])
    run_phase(PH1)

    plsc.subcore_barrier()
    pltpu.sync_copy(acc_sh.at[pl.ds(s * RPT, RPT)],
                    out_hbm.at[pl.ds(c * NP + s * RPT, RPT)])

    if with_counts:
      plsc.subcore_barrier()
      fill(rows_v, 0.0)
      zero_acc(rows_v, acc_sh, s)
      fill(rows_v, 1.0)
      plsc.subcore_barrier()

      def cstep(nch):
        def cs(j, carry):
          pltpu.sync_copy(rows_v.at[0], acc_sh.at[idx_v.at[1].at[j]],
                          add=True)
          return carry

        lax.fori_loop(0, nch, cs, 0)

      pltpu.sync_copy(edge_hbm.at[wid, :, pl.ds(0, PH0)], idx_v)
      cstep(PH0)
      pltpu.sync_copy(edge_hbm.at[wid, :, pl.ds(PH0, PH1)],
                      idx_v.at[:, pl.ds(0, PH1)])
      cstep(PH1)
      plsc.subcore_barrier()
      pltpu.sync_copy(acc_sh.at[pl.ds(s * RPT, RPT)],
                      cnt_hbm.at[pl.ds(c * NP + s * RPT, RPT)])

  out_type = [jax.ShapeDtypeStruct((NC * NP, width), jnp.float32)]
  if with_counts:
    out_type.append(jax.ShapeDtypeStruct((NC * NP, width), jnp.float32))
  return pl.kernel(
      body,
      out_type=out_type,
      mesh=mesh,
      scratch_types=[
          pltpu.VMEM((2, PH0, CH), jnp.int32),
          pltpu.VMEM((2, CH, width), jnp.float32),
          pltpu.VMEM_SHARED((NP, width), jnp.float32),
          pltpu.SemaphoreType.DMA,
      ])


_agg128c = _make_agg(128, True)
_agg128 = _make_agg(128, False)


def _pre_body(x_ref, wl_ref, wr_ref, b_ref, y_ref, r_ref):
  xb = x_ref[...]
  y_ref[...] = jnp.dot(xb, wl_ref[...], preferred_element_type=jnp.float32)
  r_ref[...] = (jnp.dot(xb, wr_ref[...], preferred_element_type=jnp.float32)
                + b_ref[...])


def _mid_body(p_ref, c_ref, r1_ref, wr_ref, b_ref, h_ref, r_ref, cm_ref):
  cm = jnp.maximum(c_ref[0, :, 0:1] + c_ref[1, :, 0:1], 1.0)
  cm_ref[...] = cm
  mean = (p_ref[0] + p_ref[1]) / cm
  h = jnp.maximum(mean + r1_ref[...], 0.0)
  h_ref[...] = h
  r_ref[...] = (jnp.dot(h, wr_ref[...], preferred_element_type=jnp.float32)
                + b_ref[...])


def _post_body(p_ref, cm_ref, r2_ref, wl_ref, wt_ref, b_ref, o_ref):
  mean = (p_ref[0] + p_ref[1]) / cm_ref[...]
  agg = jnp.dot(mean, wl_ref[...], preferred_element_type=jnp.float32)
  h = jnp.maximum(agg + r2_ref[...], 0.0)
  l0 = jnp.sum(h * wt_ref[0:1, :], axis=1, keepdims=True) + b_ref[0, 0]
  l1 = jnp.sum(h * wt_ref[1:2, :], axis=1, keepdims=True) + b_ref[0, 1]
  m = jnp.maximum(l0, l1)
  lse = m + jnp.log(jnp.exp(l0 - m) + jnp.exp(l1 - m))
  o_ref[...] = jnp.concatenate([l0 - lse, l1 - lse], axis=1)


def _full(shape):
  return pl.BlockSpec(shape, lambda i: (0,) * len(shape))


def _rows(shape):
  return pl.BlockSpec(shape, lambda i: (i,) + (0,) * (len(shape) - 1))


def _pre(x, wl, wr, b):
  return pl.pallas_call(
      _pre_body,
      grid=(NP // BR,),
      in_specs=[_rows((BR, 128)), _full((128, 128)), _full((128, 128)),
                _full((1, 128))],
      out_specs=[_rows((BR, 128)), _rows((BR, 128))],
      out_shape=[jax.ShapeDtypeStruct((NP, 128), jnp.float32)] * 2,
  )(x, wl, wr, b)


def _mid(p, cnt, r1, wr, b):
  return pl.pallas_call(
      _mid_body,
      grid=(NP // BR,),
      in_specs=[pl.BlockSpec((2, BR, 128), lambda i: (0, i, 0)),
                pl.BlockSpec((2, BR, 128), lambda i: (0, i, 0)),
                _rows((BR, 128)), _full((128, 64)), _full((1, 64))],
      out_specs=[_rows((BR, 128)), _rows((BR, 64)), _rows((BR, 1))],
      out_shape=[jax.ShapeDtypeStruct((NP, 128), jnp.float32),
                 jax.ShapeDtypeStruct((NP, 64), jnp.float32),
                 jax.ShapeDtypeStruct((NP, 1), jnp.float32)],
  )(p, cnt, r1, wr, b)


def _post(p, cm, r2, wl, wt, b):
  return pl.pallas_call(
      _post_body,
      grid=(NP // BR,),
      in_specs=[pl.BlockSpec((2, BR, 128), lambda i: (0, i, 0)),
                _rows((BR, 1)), _rows((BR, 64)), _full((128, 64)),
                _full((2, 64)), _full((1, 2))],
      out_specs=_rows((BR, 2)),
      out_shape=jax.ShapeDtypeStruct((NP, 2), jnp.float32),
  )(p, cm, r2, wl, wt, b)


def kernel(x, edge_index, W1_l, b1, W1_r, W2_l, b2, W2_r, W_lin, b_lin):
  src3 = edge_index[0].astype(jnp.int32).reshape(NW, NCH, CH)
  dst3 = edge_index[1].astype(jnp.int32).reshape(NW, NCH, CH)
  edge3 = jnp.stack([src3, dst3], axis=1)
  x_p = jnp.pad(x, ((0, NP - N), (0, 0)))

  y1, r1 = _pre(x_p, W1_l, W1_r, b1.reshape(1, -1))
  p1, cnt = _agg128c(y1, edge3)
  p1 = p1.reshape(NC, NP, 128)
  cnt = cnt.reshape(NC, NP, 128)
  h1, r2, cm = _mid(p1, cnt, r1, W2_r, b2.reshape(1, -1))
  (p2,) = _agg128(h1, edge3)
  p2 = p2.reshape(NC, NP, 128)
  out = _post(p2, cm, r2, W2_l, W_lin.T, b_lin.reshape(1, -1))
  return out[:N]


# speedup vs baseline: 1.4654x; 1.0010x over previous
import jax
import jax.numpy as jnp
from jax import lax
from jax.experimental import pallas as pl
from jax.experimental.pallas import tpu as pltpu
from jax.experimental.pallas import tpu_sc as plsc

N = 10000
E = 320000
NC, NS = 2, 16
NW = NC * NS
CH = 80
NCH = E // NW // CH
PH0, PH1 = 64, 61
NP = 10240
RPT = NP // NS
BR = 1024
mesh = plsc.VectorSubcoreMesh(core_axis_name="c", subcore_axis_name="s")


def _make_agg(width, with_counts):
  def fill(rows_v, val):
    def vstep(r, carry):
      for cc in range(8):
        rows_v[0, r, pl.ds(cc * 16, 16)] = jnp.full((16,), val, jnp.float32)
      return carry

    lax.fori_loop(0, CH, vstep, 0)

  def zero_acc(rows_v, acc_sh, s):
    def zstep(k, carry):
      pltpu.sync_copy(rows_v.at[0],
                      acc_sh.at[pl.ds(s * RPT + k * CH, CH)])
      return carry

    lax.fori_loop(0, RPT // CH, zstep, 0)

  def body(y_hbm, edge_hbm, out_hbm, *rest):
    if with_counts:
      cnt_hbm, idx_v, rows_v, acc_sh, sem = rest
    else:
      idx_v, rows_v, acc_sh, sem = rest
    c = lax.axis_index("c")
    s = lax.axis_index("s")
    wid = c * NS + s

    fill(rows_v, 0.0)
    zero_acc(rows_v, acc_sh, s)
    pltpu.sync_copy(edge_hbm.at[wid, :, pl.ds(0, PH0)], idx_v)
    src_v = idx_v.at[0]
    dst_v = idx_v.at[1]
    plsc.subcore_barrier()

    def run_phase(nch):
      pltpu.async_copy(y_hbm.at[src_v.at[0]], rows_v.at[0], sem)

      def step(j, carry):
        p = lax.rem(j, 2)
        q = 1 - p
        pltpu.make_async_copy(y_hbm.at[src_v.at[j]], rows_v.at[p],
                              sem).wait()

        @pl.when(j + 1 < nch)
        def _():
          pltpu.async_copy(y_hbm.at[src_v.at[j + 1]], rows_v.at[q], sem)

        pltpu.sync_copy(rows_v.at[p], acc_sh.at[dst_v.at[j]], add=True)
        return carry

      lax.fori_loop(0, nch, step, 0)

    run_phase(PH0)
    pltpu.sync_copy(edge_hbm.at[wid, :, pl.ds(PH0, PH1)],
                    idx_v.at[:, pl.ds(0, PH1)])
    run_phase(PH1)

    plsc.subcore_barrier()
    pltpu.sync_copy(acc_sh.at[pl.ds(s * RPT, RPT)],
                    out_hbm.at[pl.ds(c * NP + s * RPT, RPT)])

    if with_counts:
      plsc.subcore_barrier()
      fill(rows_v, 0.0)
      zero_acc(rows_v, acc_sh, s)
      fill(rows_v, 1.0)
      plsc.subcore_barrier()

      def cstep(nch):
        def cs4(k, carry):
          j = 4 * k
          for t in range(4):
            pltpu.async_copy(rows_v.at[0],
                             acc_sh.at[idx_v.at[1].at[j + t]], sem,
                             add=True)
          for t in range(4):
            pltpu.make_async_copy(rows_v.at[0],
                                  acc_sh.at[idx_v.at[1].at[j + t]],
                                  sem).wait()
          return carry

        lax.fori_loop(0, nch // 4, cs4, 0)
        for t in range(nch % 4):
          j = 4 * (nch // 4) + t
          pltpu.sync_copy(rows_v.at[0], acc_sh.at[idx_v.at[1].at[j]],
                          add=True)

      pltpu.sync_copy(edge_hbm.at[wid, :, pl.ds(0, PH0)], idx_v)
      cstep(PH0)
      pltpu.sync_copy(edge_hbm.at[wid, :, pl.ds(PH0, PH1)],
                      idx_v.at[:, pl.ds(0, PH1)])
      cstep(PH1)
      plsc.subcore_barrier()
      pltpu.sync_copy(acc_sh.at[pl.ds(s * RPT, RPT)],
                      cnt_hbm.at[pl.ds(c * NP + s * RPT, RPT)])

  out_type = [jax.ShapeDtypeStruct((NC * NP, width), jnp.float32)]
  if with_counts:
    out_type.append(jax.ShapeDtypeStruct((NC * NP, width), jnp.float32))
  return pl.kernel(
      body,
      out_type=out_type,
      mesh=mesh,
      scratch_types=[
          pltpu.VMEM((2, PH0, CH), jnp.int32),
          pltpu.VMEM((2, CH, width), jnp.float32),
          pltpu.VMEM_SHARED((NP, width), jnp.float32),
          pltpu.SemaphoreType.DMA,
      ])


_agg128c = _make_agg(128, True)
_agg128 = _make_agg(128, False)


def _pre_body(x_ref, wl_ref, wr_ref, b_ref, y_ref, r_ref):
  xb = x_ref[...]
  y_ref[...] = jnp.dot(xb, wl_ref[...], preferred_element_type=jnp.float32)
  r_ref[...] = (jnp.dot(xb, wr_ref[...], preferred_element_type=jnp.float32)
                + b_ref[...])


def _mid_body(p_ref, c_ref, r1_ref, wr_ref, b_ref, h_ref, r_ref, cm_ref):
  cm = jnp.maximum(c_ref[0, :, 0:1] + c_ref[1, :, 0:1], 1.0)
  cm_ref[...] = cm
  mean = (p_ref[0] + p_ref[1]) / cm
  h = jnp.maximum(mean + r1_ref[...], 0.0)
  h_ref[...] = h
  r_ref[...] = (jnp.dot(h, wr_ref[...], preferred_element_type=jnp.float32)
                + b_ref[...])


def _post_body(p_ref, cm_ref, r2_ref, wl_ref, wt_ref, b_ref, o_ref):
  mean = (p_ref[0] + p_ref[1]) / cm_ref[...]
  agg = jnp.dot(mean, wl_ref[...], preferred_element_type=jnp.float32)
  h = jnp.maximum(agg + r2_ref[...], 0.0)
  l0 = jnp.sum(h * wt_ref[0:1, :], axis=1, keepdims=True) + b_ref[0, 0]
  l1 = jnp.sum(h * wt_ref[1:2, :], axis=1, keepdims=True) + b_ref[0, 1]
  m = jnp.maximum(l0, l1)
  lse = m + jnp.log(jnp.exp(l0 - m) + jnp.exp(l1 - m))
  o_ref[...] = jnp.concatenate([l0 - lse, l1 - lse], axis=1)


def _full(shape):
  return pl.BlockSpec(shape, lambda i: (0,) * len(shape))


def _rows(shape):
  return pl.BlockSpec(shape, lambda i: (i,) + (0,) * (len(shape) - 1))


def _pre(x, wl, wr, b):
  return pl.pallas_call(
      _pre_body,
      grid=(NP // BR,),
      in_specs=[_rows((BR, 128)), _full((128, 128)), _full((128, 128)),
                _full((1, 128))],
      out_specs=[_rows((BR, 128)), _rows((BR, 128))],
      out_shape=[jax.ShapeDtypeStruct((NP, 128), jnp.float32)] * 2,
  )(x, wl, wr, b)


def _mid(p, cnt, r1, wr, b):
  return pl.pallas_call(
      _mid_body,
      grid=(NP // BR,),
      in_specs=[pl.BlockSpec((2, BR, 128), lambda i: (0, i, 0)),
                pl.BlockSpec((2, BR, 128), lambda i: (0, i, 0)),
                _rows((BR, 128)), _full((128, 64)), _full((1, 64))],
      out_specs=[_rows((BR, 128)), _rows((BR, 64)), _rows((BR, 1))],
      out_shape=[jax.ShapeDtypeStruct((NP, 128), jnp.float32),
                 jax.ShapeDtypeStruct((NP, 64), jnp.float32),
                 jax.ShapeDtypeStruct((NP, 1), jnp.float32)],
  )(p, cnt, r1, wr, b)


def _post(p, cm, r2, wl, wt, b):
  return pl.pallas_call(
      _post_body,
      grid=(NP // BR,),
      in_specs=[pl.BlockSpec((2, BR, 128), lambda i: (0, i, 0)),
                _rows((BR, 1)), _rows((BR, 64)), _full((128, 64)),
                _full((2, 64)), _full((1, 2))],
      out_specs=_rows((BR, 2)),
      out_shape=jax.ShapeDtypeStruct((NP, 2), jnp.float32),
  )(p, cm, r2, wl, wt, b)


def kernel(x, edge_index, W1_l, b1, W1_r, W2_l, b2, W2_r, W_lin, b_lin):
  src3 = edge_index[0].astype(jnp.int32).reshape(NW, NCH, CH)
  dst3 = edge_index[1].astype(jnp.int32).reshape(NW, NCH, CH)
  edge3 = jnp.stack([src3, dst3], axis=1)
  x_p = jnp.pad(x, ((0, NP - N), (0, 0)))

  y1, r1 = _pre(x_p, W1_l, W1_r, b1.reshape(1, -1))
  p1, cnt = _agg128c(y1, edge3)
  p1 = p1.reshape(NC, NP, 128)
  cnt = cnt.reshape(NC, NP, 128)
  h1, r2, cm = _mid(p1, cnt, r1, W2_r, b2.reshape(1, -1))
  (p2,) = _agg128(h1, edge3)
  p2 = p2.reshape(NC, NP, 128)
  out = _post(p2, cm, r2, W2_l, W_lin.T, b_lin.reshape(1, -1))
  return out[:N]
